# Initial kernel scaffold; baseline (speedup 1.0000x reference)
import jax
import jax.numpy as jnp
from jax.experimental import pallas as pl


def kernel(h, g, W, a_src, a_dst, Ws, bs, q, Wp, bp):
    raise NotImplementedError("write your pallas kernel here")


# speedup vs baseline: 1.0411x; 1.0411x over previous
import functools
import jax
import jax.numpy as jnp
from jax import lax
from jax.experimental import pallas as pl
from jax.experimental.pallas import tpu as pltpu

_N = 10000
_E = 320000
_P = 3
_DIN = 128
_H = 8
_DHID = 64
_DOUT = 16
_DSEM = 128
_ALPHA = 0.1

_NB = 10
_BN = _N // _NB
_NC = 4
_CW = 128


def _pre_body(h_ref, w_ref, as_ref, ad_ref, wh_ref, es_ref, ed_ref):
    c = pl.program_id(2)
    hb = h_ref[...]
    wh = jnp.dot(hb, w_ref[0], preferred_element_type=jnp.float32)
    wh_ref[0, 0] = wh
    es = jnp.dot(wh, as_ref[0], preferred_element_type=jnp.float32)
    ed = jnp.dot(wh, ad_ref[0], preferred_element_type=jnp.float32)

    @pl.when(c == 0)
    def _():
        es_ref[0] = es
        ed_ref[0] = ed

    @pl.when(c != 0)
    def _():
        es_ref[0] = es_ref[0] + es
        ed_ref[0] = ed_ref[0] + ed


def _pre(h, W, As, Ad):
    return pl.pallas_call(
        _pre_body,
        grid=(_P, _NB, _NC),
        in_specs=[
            pl.BlockSpec((_BN, _DIN), lambda p, i, c: (i, 0)),
            pl.BlockSpec((1, _DIN, _CW), lambda p, i, c: (p, 0, c)),
            pl.BlockSpec((1, _CW, 16), lambda p, i, c: (p, c, 0)),
            pl.BlockSpec((1, _CW, 16), lambda p, i, c: (p, c, 0)),
        ],
        out_specs=[
            pl.BlockSpec((1, 1, _BN, _CW), lambda p, i, c: (p, c, i, 0)),
            pl.BlockSpec((1, _BN, 16), lambda p, i, c: (p, i, 0)),
            pl.BlockSpec((1, _BN, 16), lambda p, i, c: (p, i, 0)),
        ],
        out_shape=[
            jax.ShapeDtypeStruct((_P, _NC, _N, _CW), jnp.float32),
            jax.ShapeDtypeStruct((_P, _N, 16), jnp.float32),
            jax.ShapeDtypeStruct((_P, _N, 16), jnp.float32),
        ],
        compiler_params=pltpu.CompilerParams(
            dimension_semantics=("parallel", "parallel", "arbitrary")),
    )(h, W, As, Ad)


def _sem_body(agg_ref, ws_ref, bs_ref, q_ref, wsum_ref):
    i = pl.program_id(0)
    acc = jnp.zeros((_P, 128), jnp.float32)
    rows = []
    for p in range(_P):
        s = jnp.zeros((_BN, _DSEM), jnp.float32)
        for c in range(_NC):
            z = agg_ref[p, c]
            z = jnp.where(z > 0, z, jnp.exp(z) - 1.0)
            s = s + jnp.dot(z, ws_ref[c],
                            preferred_element_type=jnp.float32)
        s = jnp.tanh(s + bs_ref[0][None, :])
        wp = jnp.dot(s, q_ref[...].reshape(_DSEM, 1),
                     preferred_element_type=jnp.float32)
        rows.append(jnp.full((128,), jnp.sum(wp), jnp.float32))
    acc = jnp.stack(rows)

    @pl.when(i == 0)
    def _():
        wsum_ref[...] = acc

    @pl.when(i != 0)
    def _():
        wsum_ref[...] = wsum_ref[...] + acc


def _sem(agg, Ws4, bs, q):
    return pl.pallas_call(
        _sem_body,
        grid=(_NB,),
        in_specs=[
            pl.BlockSpec((_P, _NC, _BN, _CW), lambda i: (0, 0, i, 0)),
            pl.BlockSpec((_NC, _CW, _DSEM), lambda i: (0, 0, 0)),
            pl.BlockSpec((1, _DSEM), lambda i: (0, 0)),
            pl.BlockSpec((1, _DSEM), lambda i: (0, 0)),
        ],
        out_specs=pl.BlockSpec((_P, 128), lambda i: (0, 0)),
        out_shape=jax.ShapeDtypeStruct((_P, 128), jnp.float32),
        compiler_params=pltpu.CompilerParams(
            dimension_semantics=("arbitrary",)),
    )(agg, Ws4, bs.reshape(1, _DSEM), q.reshape(1, _DSEM))


def _head_body(agg_ref, beta_ref, wp_ref, bp_ref, out_ref):
    logits = jnp.broadcast_to(bp_ref[0][None, :], (_BN, _DOUT))
    for c in range(_NC):
        zf = jnp.zeros((_BN, _CW), jnp.float32)
        for p in range(_P):
            z = agg_ref[p, c]
            z = jnp.where(z > 0, z, jnp.exp(z) - 1.0)
            zf = zf + beta_ref[p] * z
        logits = logits + jnp.dot(zf, wp_ref[c],
                                  preferred_element_type=jnp.float32)
    m = jnp.max(logits, axis=1, keepdims=True)
    sh = logits - m
    lse = jnp.log(jnp.sum(jnp.exp(sh), axis=1, keepdims=True))
    out_ref[...] = sh - lse


def _head(agg, beta, Wp4, bp):
    return pl.pallas_call(
        _head_body,
        grid=(_NB,),
        in_specs=[
            pl.BlockSpec((_P, _NC, _BN, _CW), lambda i: (0, 0, i, 0)),
            pl.BlockSpec(memory_space=pltpu.SMEM),
            pl.BlockSpec((_NC, _CW, _DOUT), lambda i: (0, 0, 0)),
            pl.BlockSpec((1, _DOUT), lambda i: (0, 0)),
        ],
        out_specs=pl.BlockSpec((_BN, _DOUT), lambda i: (i, 0)),
        out_shape=jax.ShapeDtypeStruct((_N, _DOUT), jnp.float32),
        compiler_params=pltpu.CompilerParams(
            dimension_semantics=("arbitrary",)),
    )(agg, beta, Wp4, bp.reshape(1, _DOUT))


def _edge_phase(whT, es_p, ed_p, g):
    aggs = []
    for p in range(_P):
        src = g[p, 0]
        dst = g[p, 1]
        e = es_p[p, :, :8][src] + ed_p[p, :, :8][dst]
        e = jnp.where(e > 0, e, _ALPHA * e)
        ex = jnp.exp(e)
        denom = jax.ops.segment_sum(ex, dst, num_segments=_N)
        attn = ex / (denom[dst] + 1e-9)
        Wh = jnp.moveaxis(whT[p], 0, 1).reshape(_N, _H, _DHID)
        msg = attn[:, :, None] * Wh[src]
        out = jax.ops.segment_sum(msg, dst, num_segments=_N)
        aggs.append(out.reshape(_N, _NC, _CW).swapaxes(0, 1))
    return jnp.stack(aggs)


def kernel(h, g, W, a_src, a_dst, Ws, bs, q, Wp, bp):
    mask = (jnp.arange(16)[None, :] % _H ==
            jnp.arange(_H)[:, None]).astype(jnp.float32)
    As = (a_src[:, :, :, None] * mask[None, :, None, :]).reshape(
        _P, _H * _DHID, 16)
    Ad = (a_dst[:, :, :, None] * mask[None, :, None, :]).reshape(
        _P, _H * _DHID, 16)

    whT, es_p, ed_p = _pre(h, W, As, Ad)

    agg = _edge_phase(whT, es_p, ed_p, g)

    Ws4 = Ws.reshape(_NC, _CW, _DSEM)
    wsum = _sem(agg, Ws4, bs, q)
    beta = jax.nn.softmax(wsum[:, 0] / _N)

    Wp4 = Wp.reshape(_NC, _CW, _DOUT)
    return _head(agg, beta, Wp4, bp)


# speedup vs baseline: 13.1307x; 12.6128x over previous
import functools
import jax
import jax.numpy as jnp
from jax import lax
from jax.experimental import pallas as pl
from jax.experimental.pallas import tpu as pltpu
from jax.experimental.pallas import tpu_sc as plsc

_N = 10000
_E = 320000
_P = 3
_DIN = 128
_H = 8
_DHID = 64
_DOUT = 16
_DSEM = 128
_ALPHA = 0.1

_NB = 10
_BN = _N // _NB
_NC = 4
_CW = 128


def _pre_body(h_ref, w_ref, as_ref, ad_ref, wh_ref, es_ref, ed_ref):
    c = pl.program_id(2)
    hb = h_ref[...]
    wh = jnp.dot(hb, w_ref[0], preferred_element_type=jnp.float32)
    wh_ref[0, 0] = wh
    es = jnp.dot(wh, as_ref[0], preferred_element_type=jnp.float32)
    ed = jnp.dot(wh, ad_ref[0], preferred_element_type=jnp.float32)

    @pl.when(c == 0)
    def _():
        es_ref[0] = es
        ed_ref[0] = ed

    @pl.when(c != 0)
    def _():
        es_ref[0] = es_ref[0] + es
        ed_ref[0] = ed_ref[0] + ed


def _pre(h, W, As, Ad):
    return pl.pallas_call(
        _pre_body,
        grid=(_P, _NB, _NC),
        in_specs=[
            pl.BlockSpec((_BN, _DIN), lambda p, i, c: (i, 0)),
            pl.BlockSpec((1, _DIN, _CW), lambda p, i, c: (p, 0, c)),
            pl.BlockSpec((1, _CW, 16), lambda p, i, c: (p, c, 0)),
            pl.BlockSpec((1, _CW, 16), lambda p, i, c: (p, c, 0)),
        ],
        out_specs=[
            pl.BlockSpec((1, 1, _BN, _CW), lambda p, i, c: (p, c, i, 0)),
            pl.BlockSpec((1, _BN, 16), lambda p, i, c: (p, i, 0)),
            pl.BlockSpec((1, _BN, 16), lambda p, i, c: (p, i, 0)),
        ],
        out_shape=[
            jax.ShapeDtypeStruct((_P, _NC, _N, _CW), jnp.float32),
            jax.ShapeDtypeStruct((_P, _N, 16), jnp.float32),
            jax.ShapeDtypeStruct((_P, _N, 16), jnp.float32),
        ],
        compiler_params=pltpu.CompilerParams(
            dimension_semantics=("parallel", "parallel", "arbitrary")),
    )(h, W, As, Ad)


def _sem_body(agg_ref, ws_ref, bs_ref, q_ref, wsum_ref):
    i = pl.program_id(0)
    acc = jnp.zeros((_P, 128), jnp.float32)
    rows = []
    for p in range(_P):
        s = jnp.zeros((_BN, _DSEM), jnp.float32)
        for c in range(_NC):
            z = agg_ref[p, c]
            z = jnp.where(z > 0, z, jnp.exp(z) - 1.0)
            s = s + jnp.dot(z, ws_ref[c],
                            preferred_element_type=jnp.float32)
        s = jnp.tanh(s + bs_ref[0][None, :])
        wp = jnp.dot(s, q_ref[...].reshape(_DSEM, 1),
                     preferred_element_type=jnp.float32)
        rows.append(jnp.full((128,), jnp.sum(wp), jnp.float32))
    acc = jnp.stack(rows)

    @pl.when(i == 0)
    def _():
        wsum_ref[...] = acc

    @pl.when(i != 0)
    def _():
        wsum_ref[...] = wsum_ref[...] + acc


def _sem(agg, Ws4, bs, q):
    return pl.pallas_call(
        _sem_body,
        grid=(_NB,),
        in_specs=[
            pl.BlockSpec((_P, _NC, _BN, _CW), lambda i: (0, 0, i, 0)),
            pl.BlockSpec((_NC, _CW, _DSEM), lambda i: (0, 0, 0)),
            pl.BlockSpec((1, _DSEM), lambda i: (0, 0)),
            pl.BlockSpec((1, _DSEM), lambda i: (0, 0)),
        ],
        out_specs=pl.BlockSpec((_P, 128), lambda i: (0, 0)),
        out_shape=jax.ShapeDtypeStruct((_P, 128), jnp.float32),
        compiler_params=pltpu.CompilerParams(
            dimension_semantics=("arbitrary",)),
    )(agg, Ws4, bs.reshape(1, _DSEM), q.reshape(1, _DSEM))


def _head_body(agg_ref, beta_ref, wp_ref, bp_ref, out_ref):
    logits = jnp.broadcast_to(bp_ref[0][None, :], (_BN, _DOUT))
    for c in range(_NC):
        zf = jnp.zeros((_BN, _CW), jnp.float32)
        for p in range(_P):
            z = agg_ref[p, c]
            z = jnp.where(z > 0, z, jnp.exp(z) - 1.0)
            zf = zf + beta_ref[p] * z
        logits = logits + jnp.dot(zf, wp_ref[c],
                                  preferred_element_type=jnp.float32)
    m = jnp.max(logits, axis=1, keepdims=True)
    sh = logits - m
    lse = jnp.log(jnp.sum(jnp.exp(sh), axis=1, keepdims=True))
    out_ref[...] = sh - lse


def _head(agg, beta, Wp4, bp):
    return pl.pallas_call(
        _head_body,
        grid=(_NB,),
        in_specs=[
            pl.BlockSpec((_P, _NC, _BN, _CW), lambda i: (0, 0, i, 0)),
            pl.BlockSpec(memory_space=pltpu.SMEM),
            pl.BlockSpec((_NC, _CW, _DOUT), lambda i: (0, 0, 0)),
            pl.BlockSpec((1, _DOUT), lambda i: (0, 0)),
        ],
        out_specs=pl.BlockSpec((_BN, _DOUT), lambda i: (i, 0)),
        out_shape=jax.ShapeDtypeStruct((_N, _DOUT), jnp.float32),
        compiler_params=pltpu.CompilerParams(
            dimension_semantics=("arbitrary",)),
    )(agg, beta, Wp4, bp.reshape(1, _DOUT))


_EPAD = 327680
_ER = _EPAD // 128
_NPAD = 10112
_KA = 1024
_KC = 512
_RPT = _NPAD // 16

_MESH = dict(core_axis_name="c", subcore_axis_name="s")


def _full16(v):
    return jnp.full((16,), v, jnp.int32)


def _sc_a_body(es_hbm, ed_hbm, srcr, dstr, ex_out, den_out,
               bs_v, bd_v, exb, srcv, dstv, acc, sem):
    cid = lax.axis_index("c")
    sid = lax.axis_index("s")
    w = sid * 2 + cid

    def zrow(k, c2):
        exb[k] = jnp.zeros((16,), jnp.float32)
        return c2

    lax.fori_loop(0, _RPT, zrow, 0)
    pltpu.sync_copy(exb.at[pl.ds(0, _RPT)], acc.at[pl.ds(sid * _RPT, _RPT)])
    plsc.subcore_barrier()

    def chunk(i, carry):
        base_r = w * 80 + i * 8
        base_e = w * 10240 + i * _KA
        pltpu.sync_copy(srcr.at[pl.ds(base_r, 8)], srcv)
        pltpu.sync_copy(dstr.at[pl.ds(base_r, 8)], dstv)
        hs = []
        for j in range(8):
            hs.append(pltpu.async_copy(
                es_hbm.at[srcv.at[j]], bs_v.at[pl.ds(j * 128, 128)], sem))
            hs.append(pltpu.async_copy(
                ed_hbm.at[dstv.at[j]], bd_v.at[pl.ds(j * 128, 128)], sem))
        for hh in hs:
            hh.wait()

        def row(k, c2):
            e = bs_v[k] + bd_v[k]
            e = jnp.where(e > 0, e, _ALPHA * e)
            exb[k] = jnp.exp(e)
            return c2

        lax.fori_loop(0, _KA, row, 0)
        pltpu.sync_copy(exb, ex_out.at[pl.ds(base_e, _KA)])
        for j in range(8):
            pltpu.sync_copy(exb.at[pl.ds(j * 128, 128)],
                            acc.at[dstv.at[j]], add=True)
        return carry

    lax.fori_loop(0, 10, chunk, 0)
    plsc.subcore_barrier()
    pltpu.sync_copy(acc.at[pl.ds(sid * _RPT, _RPT)],
                    den_out.at[cid, pl.ds(sid * _RPT, _RPT)])


def _sc_a(es_pad, ed_pad, srcr, dstr):
    return pl.kernel(
        _sc_a_body,
        mesh=plsc.VectorSubcoreMesh(**_MESH),
        compiler_params=pltpu.CompilerParams(use_tc_tiling_on_sc=False),
        out_type=[
            jax.ShapeDtypeStruct((_EPAD, 16), jnp.float32),
            jax.ShapeDtypeStruct((2, _NPAD, 16), jnp.float32),
        ],
        scratch_types=[
            pltpu.VMEM((_KA, 16), jnp.float32),
            pltpu.VMEM((_KA, 16), jnp.float32),
            pltpu.VMEM((_KA, 16), jnp.float32),
            pltpu.VMEM((8, 128), jnp.int32),
            pltpu.VMEM((8, 128), jnp.int32),
            pltpu.VMEM_SHARED((_NPAD, 16), jnp.float32),
            pltpu.SemaphoreType.DMA,
        ],
    )(es_pad, ed_pad, srcr, dstr)


def _sc_c_body(whs, exr3, denT3, srcr3, dstr3, agg,
               whb, exb, denb, atb, srcv, dstv, acc, sem):
    cid = lax.axis_index("c")
    sid = lax.axis_index("s")

    def job(i, carry):
        p = i // 4
        c8 = cid * 4 + (i % 4)
        t = p * 8 + c8

        def zrow(k, c2):
            for v in range(4):
                whb[k, pl.ds(v * 16, 16)] = jnp.zeros((16,), jnp.float32)
            return c2

        lax.fori_loop(0, _KC, zrow, 0)
        pltpu.sync_copy(whb, acc.at[pl.ds(sid * _RPT, _KC)])
        pltpu.sync_copy(whb.at[pl.ds(0, _RPT - _KC)],
                        acc.at[pl.ds(sid * _RPT + _KC, _RPT - _KC)])
        plsc.subcore_barrier()
        lane = jnp.full((16,), c8, jnp.int32)

        def chunk(i2, c1):
            base_r = sid * 160 + i2 * 4
            base_e = sid * 20480 + i2 * _KC
            pltpu.sync_copy(srcr3.at[p, pl.ds(base_r, 4)], srcv)
            pltpu.sync_copy(dstr3.at[p, pl.ds(base_r, 4)], dstv)
            hs = []
            for j in range(4):
                hs.append(pltpu.async_copy(
                    whs.at[t].at[srcv.at[j]],
                    whb.at[pl.ds(j * 128, 128)], sem))
                hs.append(pltpu.async_copy(
                    denT3.at[p].at[dstv.at[j]],
                    denb.at[pl.ds(j * 128, 128)], sem))
            pltpu.sync_copy(exr3.at[p, pl.ds(base_e, _KC)], exb)
            for hh in hs:
                hh.wait()

            def arow(k, c2):
                atb[k] = exb[k] / denb[k]
                return c2

            lax.fori_loop(0, _KC, arow, 0)

            def srow(k, c2):
                m0 = plsc.load_gather(atb, [_full16(k), lane])
                for v in range(4):
                    whb[k, pl.ds(v * 16, 16)] = (
                        whb[k, pl.ds(v * 16, 16)] * m0)
                return c2

            lax.fori_loop(0, _KC, srow, 0)
            for j in range(4):
                pltpu.sync_copy(whb.at[pl.ds(j * 128, 128)],
                                acc.at[dstv.at[j]], add=True)
            return c1

        lax.fori_loop(0, 40, chunk, 0)
        plsc.subcore_barrier()
        pltpu.sync_copy(acc.at[pl.ds(sid * _RPT, _RPT)],
                        agg.at[t, pl.ds(sid * _RPT, _RPT)])
        plsc.subcore_barrier()
        return carry

    lax.fori_loop(0, _P * 4, job, 0)


def _sc_c(whs, exr3, denT3, srcr3, dstr3):
    return pl.kernel(
        _sc_c_body,
        mesh=plsc.VectorSubcoreMesh(**_MESH),
        compiler_params=pltpu.CompilerParams(use_tc_tiling_on_sc=False,
                                             needs_layout_passes=False),
        out_type=jax.ShapeDtypeStruct((_P * 8, _NPAD, 64), jnp.float32),
        scratch_types=[
            pltpu.VMEM((_KC, 64), jnp.float32),
            pltpu.VMEM((_KC, 16), jnp.float32),
            pltpu.VMEM((_KC, 16), jnp.float32),
            pltpu.VMEM((_KC, 16), jnp.float32),
            pltpu.VMEM((4, 128), jnp.int32),
            pltpu.VMEM((4, 128), jnp.int32),
            pltpu.VMEM_SHARED((_NPAD, 64), jnp.float32),
            pltpu.SemaphoreType.DMA,
        ],
    )(whs, exr3, denT3, srcr3, dstr3)


def _denc_body(din_ref, out_ref):
    out_ref[0] = din_ref[0, 0] + din_ref[0, 1] + 1e-9


def _denc(dens):
    return pl.pallas_call(
        _denc_body,
        grid=(_P, _NB),
        in_specs=[pl.BlockSpec((1, 2, _BN, 16), lambda p, i: (p, 0, i, 0))],
        out_specs=pl.BlockSpec((1, _BN, 16), lambda p, i: (p, i, 0)),
        out_shape=jax.ShapeDtypeStruct((_P, _N, 16), jnp.float32),
    )(dens)


def _edge_phase_sc(whT, es_p, ed_p, g):
    pad_idx = jnp.full((_P, _EPAD - _E), _N, jnp.int32)
    srcr = jnp.concatenate([g[:, 0, :], pad_idx], axis=1).reshape(_P, _ER, 128)
    dstr = jnp.concatenate([g[:, 1, :], pad_idx], axis=1).reshape(_P, _ER, 128)
    es_pad = jnp.pad(es_p, ((0, 0), (0, _NPAD - _N), (0, 0)))
    ed_pad = jnp.pad(ed_p, ((0, 0), (0, _NPAD - _N), (0, 0)))
    wh_pad = jnp.pad(whT, ((0, 0), (0, 0), (0, _NPAD - _N), (0, 0)))
    exs, dens = [], []
    for p in range(_P):
        ex_p, den_p = _sc_a(es_pad[p], ed_pad[p], srcr[p], dstr[p])
        exs.append(ex_p)
        dens.append(den_p)
    denT = _denc(jnp.stack(dens)[:, :, :_N])
    denT_pad = jnp.pad(denT, ((0, 0), (0, _NPAD - _N), (0, 0)),
                       constant_values=1.0)

    whs = jnp.moveaxis(
        wh_pad.reshape(_P, _NC, _NPAD, 2, 64), 3, 2).reshape(
        _P * 8, _NPAD, 64)
    agg = _sc_c(whs, jnp.stack(exs), denT_pad, srcr, dstr)
    agg = jnp.moveaxis(
        agg.reshape(_P, _NC, 2, _NPAD, 64), 2, 3).reshape(
        _P, _NC, _NPAD, _CW)
    return agg[:, :, :_N, :]


def _edge_phase(whT, es_p, ed_p, g):
    aggs = []
    for p in range(_P):
        src = g[p, 0]
        dst = g[p, 1]
        e = es_p[p, :, :8][src] + ed_p[p, :, :8][dst]
        e = jnp.where(e > 0, e, _ALPHA * e)
        ex = jnp.exp(e)
        denom = jax.ops.segment_sum(ex, dst, num_segments=_N)
        attn = ex / (denom[dst] + 1e-9)
        Wh = jnp.moveaxis(whT[p], 0, 1).reshape(_N, _H, _DHID)
        msg = attn[:, :, None] * Wh[src]
        out = jax.ops.segment_sum(msg, dst, num_segments=_N)
        aggs.append(out.reshape(_N, _NC, _CW).swapaxes(0, 1))
    return jnp.stack(aggs)


def kernel(h, g, W, a_src, a_dst, Ws, bs, q, Wp, bp):
    mask = (jnp.arange(16)[None, :] % _H ==
            jnp.arange(_H)[:, None]).astype(jnp.float32)
    As = (a_src[:, :, :, None] * mask[None, :, None, :]).reshape(
        _P, _H * _DHID, 16)
    Ad = (a_dst[:, :, :, None] * mask[None, :, None, :]).reshape(
        _P, _H * _DHID, 16)

    whT, es_p, ed_p = _pre(h, W, As, Ad)

    agg = _edge_phase_sc(whT, es_p, ed_p, g)

    Ws4 = Ws.reshape(_NC, _CW, _DSEM)
    wsum = _sem(agg, Ws4, bs, q)
    beta = jax.nn.softmax(wsum[:, 0] / _N)

    Wp4 = Wp.reshape(_NC, _CW, _DOUT)
    return _head(agg, beta, Wp4, bp)


# speedup vs baseline: 14.5569x; 1.1086x over previous
import functools
import jax
import jax.numpy as jnp
from jax import lax
from jax.experimental import pallas as pl
from jax.experimental.pallas import tpu as pltpu
from jax.experimental.pallas import tpu_sc as plsc

_N = 10000
_E = 320000
_P = 3
_DIN = 128
_H = 8
_DHID = 64
_DOUT = 16
_DSEM = 128
_ALPHA = 0.1

_NB = 10
_BN = _N // _NB
_NC = 4
_CW = 128


def _pre_body(h_ref, w_ref, as_ref, ad_ref, wh_ref, es_ref, ed_ref):
    c = pl.program_id(2)
    hb = h_ref[...]
    wh = jnp.dot(hb, w_ref[0], preferred_element_type=jnp.float32)
    wh_ref[0, 0] = wh
    es = jnp.dot(wh, as_ref[0], preferred_element_type=jnp.float32)
    ed = jnp.dot(wh, ad_ref[0], preferred_element_type=jnp.float32)

    @pl.when(c == 0)
    def _():
        es_ref[0] = es
        ed_ref[0] = ed

    @pl.when(c != 0)
    def _():
        es_ref[0] = es_ref[0] + es
        ed_ref[0] = ed_ref[0] + ed


def _pre(h, W, As, Ad):
    return pl.pallas_call(
        _pre_body,
        grid=(_P, _NB, _NC),
        in_specs=[
            pl.BlockSpec((_BN, _DIN), lambda p, i, c: (i, 0)),
            pl.BlockSpec((1, _DIN, _CW), lambda p, i, c: (p, 0, c)),
            pl.BlockSpec((1, _CW, 16), lambda p, i, c: (p, c, 0)),
            pl.BlockSpec((1, _CW, 16), lambda p, i, c: (p, c, 0)),
        ],
        out_specs=[
            pl.BlockSpec((1, 1, _BN, _CW), lambda p, i, c: (p, c, i, 0)),
            pl.BlockSpec((1, _BN, 16), lambda p, i, c: (p, i, 0)),
            pl.BlockSpec((1, _BN, 16), lambda p, i, c: (p, i, 0)),
        ],
        out_shape=[
            jax.ShapeDtypeStruct((_P, _NC, _N, _CW), jnp.float32),
            jax.ShapeDtypeStruct((_P, _N, 16), jnp.float32),
            jax.ShapeDtypeStruct((_P, _N, 16), jnp.float32),
        ],
        compiler_params=pltpu.CompilerParams(
            dimension_semantics=("parallel", "parallel", "arbitrary")),
    )(h, W, As, Ad)


def _rep128(dr, c):
    r2 = dr[:, 2 * c:2 * c + 2]
    return jnp.broadcast_to(r2[:, :, None], (_BN, 2, 64)).reshape(_BN, 128)


def _sem_body(agg_ref, dr_ref, ws_ref, bs_ref, q_ref, wsum_ref):
    i = pl.program_id(0)
    acc = jnp.zeros((_P, 128), jnp.float32)
    rows = []
    for p in range(_P):
        s = jnp.zeros((_BN, _DSEM), jnp.float32)
        for c in range(_NC):
            z = agg_ref[p, c] * _rep128(dr_ref[p], c)
            z = jnp.where(z > 0, z, jnp.exp(z) - 1.0)
            s = s + jnp.dot(z, ws_ref[c],
                            preferred_element_type=jnp.float32)
        s = jnp.tanh(s + bs_ref[0][None, :])
        wp = jnp.dot(s, q_ref[...].reshape(_DSEM, 1),
                     preferred_element_type=jnp.float32)
        rows.append(jnp.full((128,), jnp.sum(wp), jnp.float32))
    acc = jnp.stack(rows)

    @pl.when(i == 0)
    def _():
        wsum_ref[...] = acc

    @pl.when(i != 0)
    def _():
        wsum_ref[...] = wsum_ref[...] + acc


def _sem(agg, denr, Ws4, bs, q):
    return pl.pallas_call(
        _sem_body,
        grid=(_NB,),
        in_specs=[
            pl.BlockSpec((_P, _NC, _BN, _CW), lambda i: (0, 0, i, 0)),
            pl.BlockSpec((_P, _BN, 16), lambda i: (0, i, 0)),
            pl.BlockSpec((_NC, _CW, _DSEM), lambda i: (0, 0, 0)),
            pl.BlockSpec((1, _DSEM), lambda i: (0, 0)),
            pl.BlockSpec((1, _DSEM), lambda i: (0, 0)),
        ],
        out_specs=pl.BlockSpec((_P, 128), lambda i: (0, 0)),
        out_shape=jax.ShapeDtypeStruct((_P, 128), jnp.float32),
        compiler_params=pltpu.CompilerParams(
            dimension_semantics=("arbitrary",)),
    )(agg, denr, Ws4, bs.reshape(1, _DSEM), q.reshape(1, _DSEM))


def _head_body(agg_ref, dr_ref, beta_ref, wp_ref, bp_ref, out_ref):
    logits = jnp.broadcast_to(bp_ref[0][None, :], (_BN, _DOUT))
    for c in range(_NC):
        zf = jnp.zeros((_BN, _CW), jnp.float32)
        for p in range(_P):
            z = agg_ref[p, c] * _rep128(dr_ref[p], c)
            z = jnp.where(z > 0, z, jnp.exp(z) - 1.0)
            zf = zf + beta_ref[p] * z
        logits = logits + jnp.dot(zf, wp_ref[c],
                                  preferred_element_type=jnp.float32)
    m = jnp.max(logits, axis=1, keepdims=True)
    sh = logits - m
    lse = jnp.log(jnp.sum(jnp.exp(sh), axis=1, keepdims=True))
    out_ref[...] = sh - lse


def _head(agg, denr, beta, Wp4, bp):
    return pl.pallas_call(
        _head_body,
        grid=(_NB,),
        in_specs=[
            pl.BlockSpec((_P, _NC, _BN, _CW), lambda i: (0, 0, i, 0)),
            pl.BlockSpec((_P, _BN, 16), lambda i: (0, i, 0)),
            pl.BlockSpec(memory_space=pltpu.SMEM),
            pl.BlockSpec((_NC, _CW, _DOUT), lambda i: (0, 0, 0)),
            pl.BlockSpec((1, _DOUT), lambda i: (0, 0)),
        ],
        out_specs=pl.BlockSpec((_BN, _DOUT), lambda i: (i, 0)),
        out_shape=jax.ShapeDtypeStruct((_N, _DOUT), jnp.float32),
        compiler_params=pltpu.CompilerParams(
            dimension_semantics=("arbitrary",)),
    )(agg, denr, beta, Wp4, bp.reshape(1, _DOUT))


_EPAD = 327680
_ER = _EPAD // 128
_NPAD = 10112
_KA = 1024
_KC = 512
_RPT = _NPAD // 16

_MESH = dict(core_axis_name="c", subcore_axis_name="s")


def _full16(v):
    return jnp.full((16,), v, jnp.int32)


def _sc_a_body(es_hbm, ed_hbm, srcr, dstr, ex_out, den_out,
               bs_v, bd_v, exb, srcv, dstv, acc, sem):
    cid = lax.axis_index("c")
    sid = lax.axis_index("s")
    w = sid * 2 + cid

    def zrow(k, c2):
        exb[k] = jnp.zeros((16,), jnp.float32)
        return c2

    lax.fori_loop(0, _RPT, zrow, 0)
    pltpu.sync_copy(exb.at[pl.ds(0, _RPT)], acc.at[pl.ds(sid * _RPT, _RPT)])
    plsc.subcore_barrier()

    def chunk(i, carry):
        base_r = w * 80 + i * 8
        base_e = w * 10240 + i * _KA
        pltpu.sync_copy(srcr.at[pl.ds(base_r, 8)], srcv)
        pltpu.sync_copy(dstr.at[pl.ds(base_r, 8)], dstv)
        hs = []
        for j in range(8):
            hs.append(pltpu.async_copy(
                es_hbm.at[srcv.at[j]], bs_v.at[pl.ds(j * 128, 128)], sem))
            hs.append(pltpu.async_copy(
                ed_hbm.at[dstv.at[j]], bd_v.at[pl.ds(j * 128, 128)], sem))
        for hh in hs:
            hh.wait()

        def row(k, c2):
            e = bs_v[k] + bd_v[k]
            e = jnp.where(e > 0, e, _ALPHA * e)
            exb[k] = jnp.exp(e)
            return c2

        lax.fori_loop(0, _KA, row, 0)
        pltpu.sync_copy(exb, ex_out.at[pl.ds(base_e, _KA)])
        for j in range(8):
            pltpu.sync_copy(exb.at[pl.ds(j * 128, 128)],
                            acc.at[dstv.at[j]], add=True)
        return carry

    lax.fori_loop(0, 10, chunk, 0)
    plsc.subcore_barrier()
    pltpu.sync_copy(acc.at[pl.ds(sid * _RPT, _RPT)],
                    den_out.at[cid, pl.ds(sid * _RPT, _RPT)])


def _sc_a(es_pad, ed_pad, srcr, dstr):
    return pl.kernel(
        _sc_a_body,
        mesh=plsc.VectorSubcoreMesh(**_MESH),
        compiler_params=pltpu.CompilerParams(use_tc_tiling_on_sc=False),
        out_type=[
            jax.ShapeDtypeStruct((_EPAD, 16), jnp.float32),
            jax.ShapeDtypeStruct((2, _NPAD, 16), jnp.float32),
        ],
        scratch_types=[
            pltpu.VMEM((_KA, 16), jnp.float32),
            pltpu.VMEM((_KA, 16), jnp.float32),
            pltpu.VMEM((_KA, 16), jnp.float32),
            pltpu.VMEM((8, 128), jnp.int32),
            pltpu.VMEM((8, 128), jnp.int32),
            pltpu.VMEM_SHARED((_NPAD, 16), jnp.float32),
            pltpu.SemaphoreType.DMA,
        ],
    )(es_pad, ed_pad, srcr, dstr)


def _sc_c_body(whs, exr3, srcr3, dstr3, agg,
               whb, exb, srcv, dstv, acc, sem):
    cid = lax.axis_index("c")
    sid = lax.axis_index("s")

    def job(i, carry):
        p = i // 4
        c8 = cid * 4 + (i % 4)
        t = p * 8 + c8

        def zrow(k, c2):
            for v in range(4):
                whb[k, pl.ds(v * 16, 16)] = jnp.zeros((16,), jnp.float32)
            return c2

        lax.fori_loop(0, _KC, zrow, 0)
        pltpu.sync_copy(whb, acc.at[pl.ds(sid * _RPT, _KC)])
        pltpu.sync_copy(whb.at[pl.ds(0, _RPT - _KC)],
                        acc.at[pl.ds(sid * _RPT + _KC, _RPT - _KC)])
        plsc.subcore_barrier()
        lane = jnp.full((16,), c8, jnp.int32)

        def chunk(i2, c1):
            base_r = sid * 160 + i2 * 4
            base_e = sid * 20480 + i2 * _KC
            pltpu.sync_copy(srcr3.at[p, pl.ds(base_r, 4)], srcv)
            pltpu.sync_copy(dstr3.at[p, pl.ds(base_r, 4)], dstv)
            hs = []
            for j in range(4):
                hs.append(pltpu.async_copy(
                    whs.at[t].at[srcv.at[j]],
                    whb.at[pl.ds(j * 128, 128)], sem))
            pltpu.sync_copy(exr3.at[p, pl.ds(base_e, _KC)], exb)
            for hh in hs:
                hh.wait()

            def srow(k, c2):
                m0 = plsc.load_gather(exb, [_full16(k), lane])
                for v in range(4):
                    whb[k, pl.ds(v * 16, 16)] = (
                        whb[k, pl.ds(v * 16, 16)] * m0)
                return c2

            lax.fori_loop(0, _KC, srow, 0)
            for j in range(4):
                pltpu.sync_copy(whb.at[pl.ds(j * 128, 128)],
                                acc.at[dstv.at[j]], add=True)
            return c1

        lax.fori_loop(0, 40, chunk, 0)
        plsc.subcore_barrier()
        pltpu.sync_copy(acc.at[pl.ds(sid * _RPT, _RPT)],
                        agg.at[t, pl.ds(sid * _RPT, _RPT)])
        plsc.subcore_barrier()
        return carry

    lax.fori_loop(0, _P * 4, job, 0)


def _sc_c(whs, exr3, srcr3, dstr3):
    return pl.kernel(
        _sc_c_body,
        mesh=plsc.VectorSubcoreMesh(**_MESH),
        compiler_params=pltpu.CompilerParams(use_tc_tiling_on_sc=False,
                                             needs_layout_passes=False),
        out_type=jax.ShapeDtypeStruct((_P * 8, _NPAD, 64), jnp.float32),
        scratch_types=[
            pltpu.VMEM((_KC, 64), jnp.float32),
            pltpu.VMEM((_KC, 16), jnp.float32),
            pltpu.VMEM((4, 128), jnp.int32),
            pltpu.VMEM((4, 128), jnp.int32),
            pltpu.VMEM_SHARED((_NPAD, 64), jnp.float32),
            pltpu.SemaphoreType.DMA,
        ],
    )(whs, exr3, srcr3, dstr3)


def _denc_body(din_ref, out_ref):
    out_ref[0] = 1.0 / (din_ref[0, 0] + din_ref[0, 1] + 1e-9)


def _denc(dens):
    return pl.pallas_call(
        _denc_body,
        grid=(_P, _NB),
        in_specs=[pl.BlockSpec((1, 2, _BN, 16), lambda p, i: (p, 0, i, 0))],
        out_specs=pl.BlockSpec((1, _BN, 16), lambda p, i: (p, i, 0)),
        out_shape=jax.ShapeDtypeStruct((_P, _N, 16), jnp.float32),
    )(dens)


def _edge_phase_sc(whT, es_p, ed_p, g):
    pad_idx = jnp.full((_P, _EPAD - _E), _N, jnp.int32)
    srcr = jnp.concatenate([g[:, 0, :], pad_idx], axis=1).reshape(_P, _ER, 128)
    dstr = jnp.concatenate([g[:, 1, :], pad_idx], axis=1).reshape(_P, _ER, 128)
    es_pad = jnp.pad(es_p, ((0, 0), (0, _NPAD - _N), (0, 0)))
    ed_pad = jnp.pad(ed_p, ((0, 0), (0, _NPAD - _N), (0, 0)))
    wh_pad = jnp.pad(whT, ((0, 0), (0, 0), (0, _NPAD - _N), (0, 0)))
    exs, dens = [], []
    for p in range(_P):
        ex_p, den_p = _sc_a(es_pad[p], ed_pad[p], srcr[p], dstr[p])
        exs.append(ex_p)
        dens.append(den_p)
    denr = _denc(jnp.stack(dens)[:, :, :_N])

    whs = jnp.moveaxis(
        wh_pad.reshape(_P, _NC, _NPAD, 2, 64), 3, 2).reshape(
        _P * 8, _NPAD, 64)
    agg = _sc_c(whs, jnp.stack(exs), srcr, dstr)
    agg = jnp.moveaxis(
        agg.reshape(_P, _NC, 2, _NPAD, 64), 2, 3).reshape(
        _P, _NC, _NPAD, _CW)
    return agg[:, :, :_N, :], denr


def _edge_phase(whT, es_p, ed_p, g):
    aggs = []
    for p in range(_P):
        src = g[p, 0]
        dst = g[p, 1]
        e = es_p[p, :, :8][src] + ed_p[p, :, :8][dst]
        e = jnp.where(e > 0, e, _ALPHA * e)
        ex = jnp.exp(e)
        denom = jax.ops.segment_sum(ex, dst, num_segments=_N)
        attn = ex / (denom[dst] + 1e-9)
        Wh = jnp.moveaxis(whT[p], 0, 1).reshape(_N, _H, _DHID)
        msg = attn[:, :, None] * Wh[src]
        out = jax.ops.segment_sum(msg, dst, num_segments=_N)
        aggs.append(out.reshape(_N, _NC, _CW).swapaxes(0, 1))
    return jnp.stack(aggs)


def kernel(h, g, W, a_src, a_dst, Ws, bs, q, Wp, bp):
    mask = (jnp.arange(16)[None, :] % _H ==
            jnp.arange(_H)[:, None]).astype(jnp.float32)
    As = (a_src[:, :, :, None] * mask[None, :, None, :]).reshape(
        _P, _H * _DHID, 16)
    Ad = (a_dst[:, :, :, None] * mask[None, :, None, :]).reshape(
        _P, _H * _DHID, 16)

    whT, es_p, ed_p = _pre(h, W, As, Ad)

    agg, denr = _edge_phase_sc(whT, es_p, ed_p, g)

    Ws4 = Ws.reshape(_NC, _CW, _DSEM)
    wsum = _sem(agg, denr, Ws4, bs, q)
    beta = jax.nn.softmax(wsum[:, 0] / _N)

    Wp4 = Wp.reshape(_NC, _CW, _DOUT)
    return _head(agg, denr, beta, Wp4, bp)


# speedup vs baseline: 15.0781x; 1.0358x over previous
import functools
import jax
import jax.numpy as jnp
from jax import lax
from jax.experimental import pallas as pl
from jax.experimental.pallas import tpu as pltpu
from jax.experimental.pallas import tpu_sc as plsc

_N = 10000
_E = 320000
_P = 3
_DIN = 128
_H = 8
_DHID = 64
_DOUT = 16
_DSEM = 128
_ALPHA = 0.1

_NB = 10
_BN = _N // _NB
_NC = 4
_CW = 128


def _pre_body(h_ref, w_ref, as_ref, ad_ref, wh_ref, es_ref, ed_ref):
    c = pl.program_id(2)
    hb = h_ref[...]
    wh = jnp.dot(hb, w_ref[0], preferred_element_type=jnp.float32)
    wh_ref[0, 0] = wh
    es = jnp.dot(wh, as_ref[0], preferred_element_type=jnp.float32)
    ed = jnp.dot(wh, ad_ref[0], preferred_element_type=jnp.float32)

    @pl.when(c == 0)
    def _():
        es_ref[0] = es
        ed_ref[0] = ed

    @pl.when(c != 0)
    def _():
        es_ref[0] = es_ref[0] + es
        ed_ref[0] = ed_ref[0] + ed


def _pre(h, W, As, Ad):
    return pl.pallas_call(
        _pre_body,
        grid=(_P, _NB, _NC),
        in_specs=[
            pl.BlockSpec((_BN, _DIN), lambda p, i, c: (i, 0)),
            pl.BlockSpec((1, _DIN, _CW), lambda p, i, c: (p, 0, c)),
            pl.BlockSpec((1, _CW, 16), lambda p, i, c: (p, c, 0)),
            pl.BlockSpec((1, _CW, 16), lambda p, i, c: (p, c, 0)),
        ],
        out_specs=[
            pl.BlockSpec((1, 1, _BN, _CW), lambda p, i, c: (p, c, i, 0)),
            pl.BlockSpec((1, _BN, 16), lambda p, i, c: (p, i, 0)),
            pl.BlockSpec((1, _BN, 16), lambda p, i, c: (p, i, 0)),
        ],
        out_shape=[
            jax.ShapeDtypeStruct((_P, _NC, _N, _CW), jnp.float32),
            jax.ShapeDtypeStruct((_P, _N, 16), jnp.float32),
            jax.ShapeDtypeStruct((_P, _N, 16), jnp.float32),
        ],
        compiler_params=pltpu.CompilerParams(
            dimension_semantics=("parallel", "parallel", "arbitrary")),
    )(h, W, As, Ad)


def _rep128(dr, c):
    r2 = dr[:, 2 * c:2 * c + 2]
    return jnp.broadcast_to(r2[:, :, None], (_BN, 2, 64)).reshape(_BN, 128)


def _sem_body(agg_ref, dr_ref, ws_ref, bs_ref, q_ref, wsum_ref):
    i = pl.program_id(0)
    acc = jnp.zeros((_P, 128), jnp.float32)
    rows = []
    for p in range(_P):
        s = jnp.zeros((_BN, _DSEM), jnp.float32)
        for c in range(_NC):
            z = agg_ref[p, c] * _rep128(dr_ref[p], c)
            z = jnp.where(z > 0, z, jnp.exp(z) - 1.0)
            s = s + jnp.dot(z, ws_ref[c],
                            preferred_element_type=jnp.float32)
        s = jnp.tanh(s + bs_ref[0][None, :])
        wp = jnp.dot(s, q_ref[...].reshape(_DSEM, 1),
                     preferred_element_type=jnp.float32)
        rows.append(jnp.full((128,), jnp.sum(wp), jnp.float32))
    acc = jnp.stack(rows)

    @pl.when(i == 0)
    def _():
        wsum_ref[...] = acc

    @pl.when(i != 0)
    def _():
        wsum_ref[...] = wsum_ref[...] + acc


def _sem(agg, denr, Ws4, bs, q):
    return pl.pallas_call(
        _sem_body,
        grid=(_NB,),
        in_specs=[
            pl.BlockSpec((_P, _NC, _BN, _CW), lambda i: (0, 0, i, 0)),
            pl.BlockSpec((_P, _BN, 16), lambda i: (0, i, 0)),
            pl.BlockSpec((_NC, _CW, _DSEM), lambda i: (0, 0, 0)),
            pl.BlockSpec((1, _DSEM), lambda i: (0, 0)),
            pl.BlockSpec((1, _DSEM), lambda i: (0, 0)),
        ],
        out_specs=pl.BlockSpec((_P, 128), lambda i: (0, 0)),
        out_shape=jax.ShapeDtypeStruct((_P, 128), jnp.float32),
        compiler_params=pltpu.CompilerParams(
            dimension_semantics=("arbitrary",)),
    )(agg, denr, Ws4, bs.reshape(1, _DSEM), q.reshape(1, _DSEM))


def _head_body(agg_ref, dr_ref, beta_ref, wp_ref, bp_ref, out_ref):
    logits = jnp.broadcast_to(bp_ref[0][None, :], (_BN, _DOUT))
    for c in range(_NC):
        zf = jnp.zeros((_BN, _CW), jnp.float32)
        for p in range(_P):
            z = agg_ref[p, c] * _rep128(dr_ref[p], c)
            z = jnp.where(z > 0, z, jnp.exp(z) - 1.0)
            zf = zf + beta_ref[p] * z
        logits = logits + jnp.dot(zf, wp_ref[c],
                                  preferred_element_type=jnp.float32)
    m = jnp.max(logits, axis=1, keepdims=True)
    sh = logits - m
    lse = jnp.log(jnp.sum(jnp.exp(sh), axis=1, keepdims=True))
    out_ref[...] = sh - lse


def _head(agg, denr, beta, Wp4, bp):
    return pl.pallas_call(
        _head_body,
        grid=(_NB,),
        in_specs=[
            pl.BlockSpec((_P, _NC, _BN, _CW), lambda i: (0, 0, i, 0)),
            pl.BlockSpec((_P, _BN, 16), lambda i: (0, i, 0)),
            pl.BlockSpec(memory_space=pltpu.SMEM),
            pl.BlockSpec((_NC, _CW, _DOUT), lambda i: (0, 0, 0)),
            pl.BlockSpec((1, _DOUT), lambda i: (0, 0)),
        ],
        out_specs=pl.BlockSpec((_BN, _DOUT), lambda i: (i, 0)),
        out_shape=jax.ShapeDtypeStruct((_N, _DOUT), jnp.float32),
        compiler_params=pltpu.CompilerParams(
            dimension_semantics=("arbitrary",)),
    )(agg, denr, beta, Wp4, bp.reshape(1, _DOUT))


_EPAD = 327680
_ER = _EPAD // 128
_NPAD = 10112
_KA = 1024
_KC = 1024
_RPT = _NPAD // 16

_MESH = dict(core_axis_name="c", subcore_axis_name="s")


def _full16(v):
    return jnp.full((16,), v, jnp.int32)


def _sc_a_body(es_hbm, ed_hbm, srcr, dstr, ex_out, den_out,
               bs_v, bd_v, exb, srcv, dstv, acc, sem):
    cid = lax.axis_index("c")
    sid = lax.axis_index("s")
    w = sid * 2 + cid

    def zrow(k, c2):
        exb[k] = jnp.zeros((16,), jnp.float32)
        return c2

    lax.fori_loop(0, _RPT, zrow, 0)
    pltpu.sync_copy(exb.at[pl.ds(0, _RPT)], acc.at[pl.ds(sid * _RPT, _RPT)])
    plsc.subcore_barrier()

    def chunk(i, carry):
        base_r = w * 80 + i * 8
        base_e = w * 10240 + i * _KA
        pltpu.sync_copy(srcr.at[pl.ds(base_r, 8)], srcv)
        pltpu.sync_copy(dstr.at[pl.ds(base_r, 8)], dstv)
        hs = []
        for j in range(8):
            hs.append(pltpu.async_copy(
                es_hbm.at[srcv.at[j]], bs_v.at[pl.ds(j * 128, 128)], sem))
            hs.append(pltpu.async_copy(
                ed_hbm.at[dstv.at[j]], bd_v.at[pl.ds(j * 128, 128)], sem))
        for hh in hs:
            hh.wait()

        def row(k, c2):
            e = bs_v[k] + bd_v[k]
            e = jnp.where(e > 0, e, _ALPHA * e)
            exb[k] = jnp.exp(e)
            return c2

        lax.fori_loop(0, _KA, row, 0)
        pltpu.sync_copy(exb, ex_out.at[pl.ds(base_e, _KA)])
        for j in range(8):
            pltpu.sync_copy(exb.at[pl.ds(j * 128, 128)],
                            acc.at[dstv.at[j]], add=True)
        return carry

    lax.fori_loop(0, 10, chunk, 0)
    plsc.subcore_barrier()
    pltpu.sync_copy(acc.at[pl.ds(sid * _RPT, _RPT)],
                    den_out.at[cid, pl.ds(sid * _RPT, _RPT)])


def _sc_a(es_pad, ed_pad, srcr, dstr):
    return pl.kernel(
        _sc_a_body,
        mesh=plsc.VectorSubcoreMesh(**_MESH),
        compiler_params=pltpu.CompilerParams(use_tc_tiling_on_sc=False),
        out_type=[
            jax.ShapeDtypeStruct((_EPAD, 16), jnp.float32),
            jax.ShapeDtypeStruct((2, _NPAD, 16), jnp.float32),
        ],
        scratch_types=[
            pltpu.VMEM((_KA, 16), jnp.float32),
            pltpu.VMEM((_KA, 16), jnp.float32),
            pltpu.VMEM((_KA, 16), jnp.float32),
            pltpu.VMEM((8, 128), jnp.int32),
            pltpu.VMEM((8, 128), jnp.int32),
            pltpu.VMEM_SHARED((_NPAD, 16), jnp.float32),
            pltpu.SemaphoreType.DMA,
        ],
    )(es_pad, ed_pad, srcr, dstr)


def _sc_c_body(whs, exr3, srcr3, dstr3, agg,
               whb, exb, srcv, dstv, acc, sem):
    cid = lax.axis_index("c")
    sid = lax.axis_index("s")

    def job(i, carry):
        p = i // 4
        c8 = cid * 4 + (i % 4)
        t = p * 8 + c8

        def zrow(k, c2):
            for v in range(4):
                whb[k, pl.ds(v * 16, 16)] = jnp.zeros((16,), jnp.float32)
            return c2

        lax.fori_loop(0, _RPT, zrow, 0)
        pltpu.sync_copy(whb.at[pl.ds(0, _RPT)], acc.at[pl.ds(sid * _RPT, _RPT)])
        plsc.subcore_barrier()
        lane = jnp.full((16,), c8, jnp.int32)

        def chunk(i2, c1):
            base_r = sid * 160 + i2 * 8
            base_e = sid * 20480 + i2 * _KC
            pltpu.sync_copy(srcr3.at[p, pl.ds(base_r, 8)], srcv)
            pltpu.sync_copy(dstr3.at[p, pl.ds(base_r, 8)], dstv)
            hs = []
            for j in range(8):
                hs.append(pltpu.async_copy(
                    whs.at[t].at[srcv.at[j]],
                    whb.at[pl.ds(j * 128, 128)], sem))
            pltpu.sync_copy(exr3.at[p, pl.ds(base_e, _KC)], exb)
            for hh in hs:
                hh.wait()

            def srow(k, c2):
                m0 = plsc.load_gather(exb, [_full16(k), lane])
                for v in range(4):
                    whb[k, pl.ds(v * 16, 16)] = (
                        whb[k, pl.ds(v * 16, 16)] * m0)
                return c2

            lax.fori_loop(0, _KC, srow, 0)
            for j in range(8):
                pltpu.sync_copy(whb.at[pl.ds(j * 128, 128)],
                                acc.at[dstv.at[j]], add=True)
            return c1

        lax.fori_loop(0, 20, chunk, 0)
        plsc.subcore_barrier()
        pltpu.sync_copy(acc.at[pl.ds(sid * _RPT, _RPT)],
                        agg.at[t, pl.ds(sid * _RPT, _RPT)])
        plsc.subcore_barrier()
        return carry

    lax.fori_loop(0, _P * 4, job, 0)


def _sc_c(whs, exr3, srcr3, dstr3):
    return pl.kernel(
        _sc_c_body,
        mesh=plsc.VectorSubcoreMesh(**_MESH),
        compiler_params=pltpu.CompilerParams(use_tc_tiling_on_sc=False,
                                             needs_layout_passes=False),
        out_type=jax.ShapeDtypeStruct((_P * 8, _NPAD, 64), jnp.float32),
        scratch_types=[
            pltpu.VMEM((_KC, 64), jnp.float32),
            pltpu.VMEM((_KC, 16), jnp.float32),
            pltpu.VMEM((8, 128), jnp.int32),
            pltpu.VMEM((8, 128), jnp.int32),
            pltpu.VMEM_SHARED((_NPAD, 64), jnp.float32),
            pltpu.SemaphoreType.DMA,
        ],
    )(whs, exr3, srcr3, dstr3)


def _denc_body(din_ref, out_ref):
    out_ref[0] = 1.0 / (din_ref[0, 0] + din_ref[0, 1] + 1e-9)


def _denc(dens):
    return pl.pallas_call(
        _denc_body,
        grid=(_P, _NB),
        in_specs=[pl.BlockSpec((1, 2, _BN, 16), lambda p, i: (p, 0, i, 0))],
        out_specs=pl.BlockSpec((1, _BN, 16), lambda p, i: (p, i, 0)),
        out_shape=jax.ShapeDtypeStruct((_P, _N, 16), jnp.float32),
    )(dens)


def _edge_phase_sc(whT, es_p, ed_p, g):
    pad_idx = jnp.full((_P, _EPAD - _E), _N, jnp.int32)
    srcr = jnp.concatenate([g[:, 0, :], pad_idx], axis=1).reshape(_P, _ER, 128)
    dstr = jnp.concatenate([g[:, 1, :], pad_idx], axis=1).reshape(_P, _ER, 128)
    es_pad = jnp.pad(es_p, ((0, 0), (0, _NPAD - _N), (0, 0)))
    ed_pad = jnp.pad(ed_p, ((0, 0), (0, _NPAD - _N), (0, 0)))
    wh_pad = jnp.pad(whT, ((0, 0), (0, 0), (0, _NPAD - _N), (0, 0)))
    exs, dens = [], []
    for p in range(_P):
        ex_p, den_p = _sc_a(es_pad[p], ed_pad[p], srcr[p], dstr[p])
        exs.append(ex_p)
        dens.append(den_p)
    denr = _denc(jnp.stack(dens)[:, :, :_N])

    whs = jnp.moveaxis(
        wh_pad.reshape(_P, _NC, _NPAD, 2, 64), 3, 2).reshape(
        _P * 8, _NPAD, 64)
    agg = _sc_c(whs, jnp.stack(exs), srcr, dstr)
    agg = jnp.moveaxis(
        agg.reshape(_P, _NC, 2, _NPAD, 64), 2, 3).reshape(
        _P, _NC, _NPAD, _CW)
    return agg[:, :, :_N, :], denr


def _edge_phase(whT, es_p, ed_p, g):
    aggs = []
    for p in range(_P):
        src = g[p, 0]
        dst = g[p, 1]
        e = es_p[p, :, :8][src] + ed_p[p, :, :8][dst]
        e = jnp.where(e > 0, e, _ALPHA * e)
        ex = jnp.exp(e)
        denom = jax.ops.segment_sum(ex, dst, num_segments=_N)
        attn = ex / (denom[dst] + 1e-9)
        Wh = jnp.moveaxis(whT[p], 0, 1).reshape(_N, _H, _DHID)
        msg = attn[:, :, None] * Wh[src]
        out = jax.ops.segment_sum(msg, dst, num_segments=_N)
        aggs.append(out.reshape(_N, _NC, _CW).swapaxes(0, 1))
    return jnp.stack(aggs)


def kernel(h, g, W, a_src, a_dst, Ws, bs, q, Wp, bp):
    mask = (jnp.arange(16)[None, :] % _H ==
            jnp.arange(_H)[:, None]).astype(jnp.float32)
    As = (a_src[:, :, :, None] * mask[None, :, None, :]).reshape(
        _P, _H * _DHID, 16)
    Ad = (a_dst[:, :, :, None] * mask[None, :, None, :]).reshape(
        _P, _H * _DHID, 16)

    whT, es_p, ed_p = _pre(h, W, As, Ad)

    agg, denr = _edge_phase_sc(whT, es_p, ed_p, g)

    Ws4 = Ws.reshape(_NC, _CW, _DSEM)
    wsum = _sem(agg, denr, Ws4, bs, q)
    beta = jax.nn.softmax(wsum[:, 0] / _N)

    Wp4 = Wp.reshape(_NC, _CW, _DOUT)
    return _head(agg, denr, beta, Wp4, bp)


# speedup vs baseline: 15.6142x; 1.0356x over previous
import functools
import jax
import jax.numpy as jnp
from jax import lax
from jax.experimental import pallas as pl
from jax.experimental.pallas import tpu as pltpu
from jax.experimental.pallas import tpu_sc as plsc

_N = 10000
_E = 320000
_P = 3
_DIN = 128
_H = 8
_DHID = 64
_DOUT = 16
_DSEM = 128
_ALPHA = 0.1

_NB = 10
_BN = _N // _NB
_NC = 4
_CW = 128


def _pre_body(h_ref, w_ref, as_ref, ad_ref, wh_ref, es_ref, ed_ref):
    c = pl.program_id(2)
    hb = h_ref[...]
    wh = jnp.dot(hb, w_ref[0], preferred_element_type=jnp.float32)
    wh_ref[0, 0] = wh
    es = jnp.dot(wh, as_ref[0], preferred_element_type=jnp.float32)
    ed = jnp.dot(wh, ad_ref[0], preferred_element_type=jnp.float32)

    @pl.when(c == 0)
    def _():
        es_ref[0] = es
        ed_ref[0] = ed

    @pl.when(c != 0)
    def _():
        es_ref[0] = es_ref[0] + es
        ed_ref[0] = ed_ref[0] + ed


def _pre(h, W, As, Ad):
    return pl.pallas_call(
        _pre_body,
        grid=(_P, _NB, _NC),
        in_specs=[
            pl.BlockSpec((_BN, _DIN), lambda p, i, c: (i, 0)),
            pl.BlockSpec((1, _DIN, _CW), lambda p, i, c: (p, 0, c)),
            pl.BlockSpec((1, _CW, 16), lambda p, i, c: (p, c, 0)),
            pl.BlockSpec((1, _CW, 16), lambda p, i, c: (p, c, 0)),
        ],
        out_specs=[
            pl.BlockSpec((1, 1, _BN, _CW), lambda p, i, c: (p, c, i, 0)),
            pl.BlockSpec((1, _BN, 16), lambda p, i, c: (p, i, 0)),
            pl.BlockSpec((1, _BN, 16), lambda p, i, c: (p, i, 0)),
        ],
        out_shape=[
            jax.ShapeDtypeStruct((_P, _NC, _N, _CW), jnp.float32),
            jax.ShapeDtypeStruct((_P, _N, 16), jnp.float32),
            jax.ShapeDtypeStruct((_P, _N, 16), jnp.float32),
        ],
        compiler_params=pltpu.CompilerParams(
            dimension_semantics=("parallel", "parallel", "arbitrary")),
    )(h, W, As, Ad)


def _sem_body(agg_ref, dr_ref, ws_ref, bs_ref, q_ref, wsum_ref):
    i = pl.program_id(0)
    acc = jnp.zeros((_P, 128), jnp.float32)
    rows = []
    for p in range(_P):
        s = jnp.zeros((_BN, _DSEM), jnp.float32)
        for c in range(8):
            r = jnp.broadcast_to(dr_ref[p][:, c:c + 1], (_BN, 64))
            z = agg_ref[p * 8 + c] * r
            z = jnp.where(z > 0, z, jnp.exp(z) - 1.0)
            s = s + jnp.dot(z, ws_ref[c],
                            preferred_element_type=jnp.float32)
        s = jnp.tanh(s + bs_ref[0][None, :])
        wp = jnp.dot(s, q_ref[...].reshape(_DSEM, 1),
                     preferred_element_type=jnp.float32)
        rows.append(jnp.full((128,), jnp.sum(wp), jnp.float32))
    acc = jnp.stack(rows)

    @pl.when(i == 0)
    def _():
        wsum_ref[...] = acc

    @pl.when(i != 0)
    def _():
        wsum_ref[...] = wsum_ref[...] + acc


def _sem(agg, denr, Ws4, bs, q):
    return pl.pallas_call(
        _sem_body,
        grid=(_NB,),
        in_specs=[
            pl.BlockSpec((_P * 8, _BN, 64), lambda i: (0, i, 0)),
            pl.BlockSpec((_P, _BN, 16), lambda i: (0, i, 0)),
            pl.BlockSpec((8, 64, _DSEM), lambda i: (0, 0, 0)),
            pl.BlockSpec((1, _DSEM), lambda i: (0, 0)),
            pl.BlockSpec((1, _DSEM), lambda i: (0, 0)),
        ],
        out_specs=pl.BlockSpec((_P, 128), lambda i: (0, 0)),
        out_shape=jax.ShapeDtypeStruct((_P, 128), jnp.float32),
        compiler_params=pltpu.CompilerParams(
            dimension_semantics=("arbitrary",)),
    )(agg, denr, Ws4, bs.reshape(1, _DSEM), q.reshape(1, _DSEM))


def _head_body(agg_ref, dr_ref, beta_ref, wp_ref, bp_ref, out_ref):
    logits = jnp.broadcast_to(bp_ref[0][None, :], (_BN, _DOUT))
    for c in range(8):
        zf = jnp.zeros((_BN, 64), jnp.float32)
        for p in range(_P):
            r = jnp.broadcast_to(dr_ref[p][:, c:c + 1], (_BN, 64))
            z = agg_ref[p * 8 + c] * r
            z = jnp.where(z > 0, z, jnp.exp(z) - 1.0)
            zf = zf + beta_ref[p] * z
        logits = logits + jnp.dot(zf, wp_ref[c],
                                  preferred_element_type=jnp.float32)
    m = jnp.max(logits, axis=1, keepdims=True)
    sh = logits - m
    lse = jnp.log(jnp.sum(jnp.exp(sh), axis=1, keepdims=True))
    out_ref[...] = sh - lse


def _head(agg, denr, beta, Wp4, bp):
    return pl.pallas_call(
        _head_body,
        grid=(_NB,),
        in_specs=[
            pl.BlockSpec((_P * 8, _BN, 64), lambda i: (0, i, 0)),
            pl.BlockSpec((_P, _BN, 16), lambda i: (0, i, 0)),
            pl.BlockSpec(memory_space=pltpu.SMEM),
            pl.BlockSpec((8, 64, _DOUT), lambda i: (0, 0, 0)),
            pl.BlockSpec((1, _DOUT), lambda i: (0, 0)),
        ],
        out_specs=pl.BlockSpec((_BN, _DOUT), lambda i: (i, 0)),
        out_shape=jax.ShapeDtypeStruct((_N, _DOUT), jnp.float32),
        compiler_params=pltpu.CompilerParams(
            dimension_semantics=("arbitrary",)),
    )(agg, denr, beta, Wp4, bp.reshape(1, _DOUT))


_EPAD = 327680
_ER = _EPAD // 128
_NPAD = 10112
_KA = 1024
_KC = 1024
_RPT = _NPAD // 16

_MESH = dict(core_axis_name="c", subcore_axis_name="s")


def _full16(v):
    return jnp.full((16,), v, jnp.int32)


def _sc_a_body(es_hbm, ed_hbm, srcr, dstr, ex_out, den_out,
               bs_v, bd_v, exb, srcv, dstv, acc, sem):
    cid = lax.axis_index("c")
    sid = lax.axis_index("s")
    w = sid * 2 + cid

    def zrow(k, c2):
        exb[k] = jnp.zeros((16,), jnp.float32)
        return c2

    lax.fori_loop(0, _RPT, zrow, 0)
    pltpu.sync_copy(exb.at[pl.ds(0, _RPT)], acc.at[pl.ds(sid * _RPT, _RPT)])
    plsc.subcore_barrier()

    def chunk(i, carry):
        base_r = w * 80 + i * 8
        base_e = w * 10240 + i * _KA
        pltpu.sync_copy(srcr.at[pl.ds(base_r, 8)], srcv)
        pltpu.sync_copy(dstr.at[pl.ds(base_r, 8)], dstv)
        hs = []
        for j in range(8):
            hs.append(pltpu.async_copy(
                es_hbm.at[srcv.at[j]], bs_v.at[pl.ds(j * 128, 128)], sem))
            hs.append(pltpu.async_copy(
                ed_hbm.at[dstv.at[j]], bd_v.at[pl.ds(j * 128, 128)], sem))
        for hh in hs:
            hh.wait()

        def row(k, c2):
            e = bs_v[k] + bd_v[k]
            e = jnp.where(e > 0, e, _ALPHA * e)
            exb[k] = jnp.exp(e)
            return c2

        lax.fori_loop(0, _KA, row, 0)
        pltpu.sync_copy(exb, ex_out.at[pl.ds(base_e, _KA)])
        for j in range(8):
            pltpu.sync_copy(exb.at[pl.ds(j * 128, 128)],
                            acc.at[dstv.at[j]], add=True)
        return carry

    lax.fori_loop(0, 10, chunk, 0)
    plsc.subcore_barrier()
    pltpu.sync_copy(acc.at[pl.ds(sid * _RPT, _RPT)],
                    den_out.at[cid, pl.ds(sid * _RPT, _RPT)])


def _sc_a(es_pad, ed_pad, srcr, dstr):
    return pl.kernel(
        _sc_a_body,
        mesh=plsc.VectorSubcoreMesh(**_MESH),
        compiler_params=pltpu.CompilerParams(use_tc_tiling_on_sc=False),
        out_type=[
            jax.ShapeDtypeStruct((_EPAD, 16), jnp.float32),
            jax.ShapeDtypeStruct((2, _NPAD, 16), jnp.float32),
        ],
        scratch_types=[
            pltpu.VMEM((_KA, 16), jnp.float32),
            pltpu.VMEM((_KA, 16), jnp.float32),
            pltpu.VMEM((_KA, 16), jnp.float32),
            pltpu.VMEM((8, 128), jnp.int32),
            pltpu.VMEM((8, 128), jnp.int32),
            pltpu.VMEM_SHARED((_NPAD, 16), jnp.float32),
            pltpu.SemaphoreType.DMA,
        ],
    )(es_pad, ed_pad, srcr, dstr)


def _sc_c_body(whs, exr3, srcr3, dstr3, agg,
               whb, exb, srcv, dstv, acc, sem):
    cid = lax.axis_index("c")
    sid = lax.axis_index("s")

    def job(i, carry):
        p = i // 4
        c8 = cid * 4 + (i % 4)
        t = p * 8 + c8

        def zrow(k, c2):
            for v in range(4):
                whb[k, pl.ds(v * 16, 16)] = jnp.zeros((16,), jnp.float32)
            return c2

        lax.fori_loop(0, _RPT, zrow, 0)
        pltpu.sync_copy(whb.at[pl.ds(0, _RPT)], acc.at[pl.ds(sid * _RPT, _RPT)])
        plsc.subcore_barrier()
        lane = jnp.full((16,), c8, jnp.int32)

        def chunk(i2, c1):
            base_r = sid * 160 + i2 * 8
            base_e = sid * 20480 + i2 * _KC
            pltpu.sync_copy(srcr3.at[p, pl.ds(base_r, 8)], srcv)
            pltpu.sync_copy(dstr3.at[p, pl.ds(base_r, 8)], dstv)
            hs = []
            for j in range(8):
                hs.append(pltpu.async_copy(
                    whs.at[t].at[srcv.at[j]],
                    whb.at[pl.ds(j * 128, 128)], sem))
            pltpu.sync_copy(exr3.at[p, pl.ds(base_e, _KC)], exb)
            for hh in hs:
                hh.wait()

            def srow(k, c2):
                m0 = plsc.load_gather(exb, [_full16(k), lane])
                for v in range(4):
                    whb[k, pl.ds(v * 16, 16)] = (
                        whb[k, pl.ds(v * 16, 16)] * m0)
                return c2

            lax.fori_loop(0, _KC, srow, 0)
            for j in range(8):
                pltpu.sync_copy(whb.at[pl.ds(j * 128, 128)],
                                acc.at[dstv.at[j]], add=True)
            return c1

        lax.fori_loop(0, 20, chunk, 0)
        plsc.subcore_barrier()
        pltpu.sync_copy(acc.at[pl.ds(sid * _RPT, _RPT)],
                        agg.at[t, pl.ds(sid * _RPT, _RPT)])
        plsc.subcore_barrier()
        return carry

    lax.fori_loop(0, _P * 4, job, 0)


def _sc_c(whs, exr3, srcr3, dstr3):
    return pl.kernel(
        _sc_c_body,
        mesh=plsc.VectorSubcoreMesh(**_MESH),
        compiler_params=pltpu.CompilerParams(use_tc_tiling_on_sc=False,
                                             needs_layout_passes=False),
        out_type=jax.ShapeDtypeStruct((_P * 8, _NPAD, 64), jnp.float32),
        scratch_types=[
            pltpu.VMEM((_KC, 64), jnp.float32),
            pltpu.VMEM((_KC, 16), jnp.float32),
            pltpu.VMEM((8, 128), jnp.int32),
            pltpu.VMEM((8, 128), jnp.int32),
            pltpu.VMEM_SHARED((_NPAD, 64), jnp.float32),
            pltpu.SemaphoreType.DMA,
        ],
    )(whs, exr3, srcr3, dstr3)


def _denc_body(din_ref, out_ref):
    out_ref[0] = 1.0 / (din_ref[0, 0] + din_ref[0, 1] + 1e-9)


def _denc(dens):
    return pl.pallas_call(
        _denc_body,
        grid=(_P, _NB),
        in_specs=[pl.BlockSpec((1, 2, _BN, 16), lambda p, i: (p, 0, i, 0))],
        out_specs=pl.BlockSpec((1, _BN, 16), lambda p, i: (p, i, 0)),
        out_shape=jax.ShapeDtypeStruct((_P, _N, 16), jnp.float32),
    )(dens)


def _edge_phase_sc(whT, es_p, ed_p, g):
    pad_idx = jnp.full((_P, _EPAD - _E), _N, jnp.int32)
    srcr = jnp.concatenate([g[:, 0, :], pad_idx], axis=1).reshape(_P, _ER, 128)
    dstr = jnp.concatenate([g[:, 1, :], pad_idx], axis=1).reshape(_P, _ER, 128)
    es_pad = jnp.pad(es_p, ((0, 0), (0, _NPAD - _N), (0, 0)))
    ed_pad = jnp.pad(ed_p, ((0, 0), (0, _NPAD - _N), (0, 0)))
    wh_pad = jnp.pad(whT, ((0, 0), (0, 0), (0, _NPAD - _N), (0, 0)))
    exs, dens = [], []
    for p in range(_P):
        ex_p, den_p = _sc_a(es_pad[p], ed_pad[p], srcr[p], dstr[p])
        exs.append(ex_p)
        dens.append(den_p)
    denr = _denc(jnp.stack(dens)[:, :, :_N])

    whs = jnp.moveaxis(
        wh_pad.reshape(_P, _NC, _NPAD, 2, 64), 3, 2).reshape(
        _P * 8, _NPAD, 64)
    agg = _sc_c(whs, jnp.stack(exs), srcr, dstr)
    return agg, denr


def _edge_phase(whT, es_p, ed_p, g):
    aggs = []
    for p in range(_P):
        src = g[p, 0]
        dst = g[p, 1]
        e = es_p[p, :, :8][src] + ed_p[p, :, :8][dst]
        e = jnp.where(e > 0, e, _ALPHA * e)
        ex = jnp.exp(e)
        denom = jax.ops.segment_sum(ex, dst, num_segments=_N)
        attn = ex / (denom[dst] + 1e-9)
        Wh = jnp.moveaxis(whT[p], 0, 1).reshape(_N, _H, _DHID)
        msg = attn[:, :, None] * Wh[src]
        out = jax.ops.segment_sum(msg, dst, num_segments=_N)
        aggs.append(out.reshape(_N, _NC, _CW).swapaxes(0, 1))
    return jnp.stack(aggs)


def kernel(h, g, W, a_src, a_dst, Ws, bs, q, Wp, bp):
    mask = (jnp.arange(16)[None, :] % _H ==
            jnp.arange(_H)[:, None]).astype(jnp.float32)
    As = (a_src[:, :, :, None] * mask[None, :, None, :]).reshape(
        _P, _H * _DHID, 16)
    Ad = (a_dst[:, :, :, None] * mask[None, :, None, :]).reshape(
        _P, _H * _DHID, 16)

    whT, es_p, ed_p = _pre(h, W, As, Ad)

    agg, denr = _edge_phase_sc(whT, es_p, ed_p, g)

    Ws4 = Ws.reshape(8, 64, _DSEM)
    wsum = _sem(agg, denr, Ws4, bs, q)
    beta = jax.nn.softmax(wsum[:, 0] / _N)

    Wp4 = Wp.reshape(8, 64, _DOUT)
    return _head(agg, denr, beta, Wp4, bp)
